# Initial kernel scaffold; baseline (speedup 1.0000x reference)
import jax
import jax.numpy as jnp
from jax.experimental import pallas as pl


def kernel(x, embedding):
    raise NotImplementedError("write your pallas kernel here")


# speedup vs baseline: 1.4534x; 1.4534x over previous
import functools

import jax
import jax.numpy as jnp
from jax import lax
from jax.experimental import pallas as pl
from jax.experimental.pallas import tpu as pltpu
from jax.experimental.pallas import tpu_sc as plsc

DIM = 32
BATCH = 16384
FIELDS = 26

NC = 2
NS = 16
NW = NC * NS
B = BATCH * FIELDS
BPW = B // NW
G = 128
CHUNK = 3328
KG = CHUNK // G
NCHUNK = BPW // CHUNK


def _sc_gather(idx, table):
    mesh = plsc.VectorSubcoreMesh(core_axis_name="c", subcore_axis_name="s")

    @functools.partial(
        pl.kernel,
        mesh=mesh,
        out_type=jax.ShapeDtypeStruct((B, DIM), jnp.float32),
        scratch_types=[
            pltpu.VMEM((CHUNK,), jnp.int32),
            pltpu.VMEM((CHUNK, DIM), jnp.float32),
            pltpu.SemaphoreType.DMA,
        ],
        compiler_params=pltpu.CompilerParams(use_tc_tiling_on_sc=False),
    )
    def k(idx_hbm, table_hbm, out_hbm, idx_v, rows_v, sem):
        wid = lax.axis_index("s") * NC + lax.axis_index("c")
        base = wid * BPW
        for c in range(NCHUNK):
            off = base + c * CHUNK
            pltpu.sync_copy(idx_hbm.at[pl.ds(off, CHUNK)], idx_v)
            copies = []
            for j in range(KG):
                cp = pltpu.make_async_copy(
                    table_hbm.at[idx_v.at[pl.ds(j * G, G)]],
                    rows_v.at[pl.ds(j * G, G)],
                    sem,
                )
                cp.start()
                copies.append(cp)
            for cp in copies:
                cp.wait()
            pltpu.sync_copy(rows_v, out_hbm.at[pl.ds(off, CHUNK)])

    return k(idx, table)


def kernel(x, embedding):
    idx = x.reshape(-1).astype(jnp.int32)
    out = _sc_gather(idx, embedding)
    return out.reshape(BATCH, FIELDS, DIM)
